# baseline (speedup 1.0000x reference)
import functools

import jax
import jax.numpy as jnp
from jax import lax
from jax.experimental import pallas as pl
from jax.experimental.pallas import tpu as pltpu
from jax.experimental.pallas import tpu_sc as plsc

N = 10000
NPAD = 10240
D = 128
H = 16
NCLS = 40
E = 320000
EPAD = 327680
CHUNK = 128
NWORKERS = 32
NCHUNK = EPAD // (NWORKERS * CHUNK)
NCH0 = 48
NCH1 = 112
NCHMAX = max(NCH0, NCH1)
RPT = NPAD // 16

_mesh = plsc.VectorSubcoreMesh(core_axis_name="c", subcore_axis_name="s")


@functools.partial(
    pl.kernel,
    mesh=_mesh,
    out_type=jax.ShapeDtypeStruct((2, NPAD), jnp.float32),
    scratch_types=[
        pltpu.VMEM((NCHMAX, CHUNK), jnp.int32),
        pltpu.VMEM((CHUNK,), jnp.float32),
        pltpu.VMEM((RPT,), jnp.float32),
        pltpu.VMEM_SHARED((NPAD,), jnp.float32),
        pltpu.SemaphoreType.DMA,
    ],
)
def _deg_kernel(dst_hbm, out_hbm, idx_v, ones_v, zbuf_v, deg_sh, dsem):
    ci = lax.axis_index("c")
    si = lax.axis_index("s")
    nch = jnp.where(ci == 0, NCH0, NCH1)
    base = jnp.where(ci == 0, si * NCH0, 16 * NCH0 + si * NCH1)

    def fill_ones(i, _):
        ones_v[pl.ds(i * 16, 16)] = jnp.ones((16,), jnp.float32)
        return 0

    lax.fori_loop(0, CHUNK // 16, fill_ones, 0)

    def fill_zeros(i, _):
        zbuf_v[pl.ds(i * 16, 16)] = jnp.zeros((16,), jnp.float32)
        return 0

    lax.fori_loop(0, RPT // 16, fill_zeros, 0)
    pltpu.sync_copy(zbuf_v, deg_sh.at[pl.ds(si * RPT, RPT)])
    plsc.subcore_barrier()

    pltpu.sync_copy(dst_hbm.at[pl.ds(base, NCHMAX)], idx_v)

    def group(g, _):
        for b in range(16):
            pltpu.async_copy(ones_v, deg_sh.at[idx_v.at[g * 16 + b]], dsem,
                             add=True)
        for b in range(16):
            pltpu.make_async_copy(ones_v, deg_sh.at[idx_v.at[g * 16 + b]],
                                  dsem).wait()
        return 0

    lax.fori_loop(0, nch // 16, group, 0)
    plsc.subcore_barrier()
    pltpu.sync_copy(deg_sh.at[pl.ds(si * RPT, RPT)],
                    out_hbm.at[ci, pl.ds(si * RPT, RPT)])


@functools.partial(
    pl.kernel,
    mesh=_mesh,
    compiler_params=pltpu.CompilerParams(use_tc_tiling_on_sc=False),
    out_type=jax.ShapeDtypeStruct((2, NPAD, H), jnp.float32),
    scratch_types=[
        pltpu.VMEM((NCHMAX, CHUNK), jnp.int32),
        pltpu.VMEM((NCHMAX, CHUNK), jnp.int32),
        pltpu.VMEM((16, CHUNK, H), jnp.float32),
        pltpu.VMEM((CHUNK, H), jnp.float32),
        pltpu.VMEM_SHARED((NPAD, H), jnp.float32),
        [pltpu.SemaphoreType.DMA] * 8,
        [pltpu.SemaphoreType.DMA] * 8,
    ],
)
def _agg_kernel(y_hbm, src_hbm, dst_hbm, out_hbm,
                sidx, didx, rows, zbuf, acc_sh, gsem, ssem):
    ci = lax.axis_index("c")
    si = lax.axis_index("s")
    nch = jnp.where(ci == 0, NCH0, NCH1)
    base = jnp.where(ci == 0, si * NCH0, 16 * NCH0 + si * NCH1)

    idx_cp0 = pltpu.async_copy(src_hbm.at[pl.ds(base, NCHMAX)], sidx,
                               gsem[0])
    idx_cp1 = pltpu.async_copy(dst_hbm.at[pl.ds(base, NCHMAX)], didx,
                               gsem[1])

    def fill_zeros(i, _):
        zbuf[i, :] = jnp.zeros((16,), jnp.float32)
        return 0

    lax.fori_loop(0, CHUNK, fill_zeros, 0)

    def zero_slice(i, _):
        pltpu.sync_copy(zbuf, acc_sh.at[pl.ds(si * RPT + i * CHUNK, CHUNK)])
        return 0

    lax.fori_loop(0, RPT // CHUNK, zero_slice, 0)
    idx_cp0.wait()
    idx_cp1.wait()
    plsc.subcore_barrier()

    for b in range(8):
        pltpu.async_copy(y_hbm.at[sidx.at[b]], rows.at[b], gsem[b])

    def group(g, _):
        for b in range(16):
            k = g * 16 + b
            sb = b % 8

            @pl.when(k >= 8)
            def _wait_scatter():
                pltpu.make_async_copy(rows.at[(b + 8) % 16],
                                      acc_sh.at[didx.at[k - 8]],
                                      ssem[sb]).wait()

            pltpu.make_async_copy(y_hbm.at[sidx.at[k]], rows.at[b],
                                  gsem[sb]).wait()

            @pl.when(k + 8 < nch)
            def _issue_gather():
                pltpu.async_copy(y_hbm.at[sidx.at[k + 8]],
                                 rows.at[(b + 8) % 16], gsem[sb])

            pltpu.async_copy(rows.at[b], acc_sh.at[didx.at[k]], ssem[sb],
                             add=True)
        return 0

    lax.fori_loop(0, nch // 16, group, 0)
    for b in range(8):
        k = nch - 8 + b
        pltpu.make_async_copy(rows.at[(8 + b) % 16], acc_sh.at[didx.at[k]],
                              ssem[b]).wait()
    plsc.subcore_barrier()
    pltpu.sync_copy(acc_sh.at[pl.ds(si * RPT, RPT)],
                    out_hbm.at[ci, pl.ds(si * RPT, RPT)])


def _tc1_body(x_ref, w1_ref, degp_ref, y1_ref, dinv_ref):
    deg = degp_ref[0, :] + degp_ref[1, :] + 1.0
    dinv = lax.rsqrt(deg)
    dinv_ref[...] = dinv
    xw = jnp.dot(x_ref[...], w1_ref[...], preferred_element_type=jnp.float32)
    y1_ref[:N, :] = xw * dinv[:N, None]
    y1_ref[N:, :] = jnp.zeros((NPAD - N, H), jnp.float32)


_tc1 = pl.pallas_call(
    _tc1_body,
    out_shape=(jax.ShapeDtypeStruct((NPAD, H), jnp.float32),
               jax.ShapeDtypeStruct((NPAD,), jnp.float32)),
)


def _tc2_body(aggp_ref, y1_ref, dinv_ref, b1_ref, y2_ref):
    agg = aggp_ref[0] + aggp_ref[1] + y1_ref[...]
    dinv = dinv_ref[...][:, None]
    h = jnp.maximum(agg * dinv + b1_ref[...][None, :], 0.0)
    y2_ref[...] = h * dinv


_tc2 = pl.pallas_call(
    _tc2_body,
    out_shape=jax.ShapeDtypeStruct((NPAD, H), jnp.float32),
)


def _tc3_body(aggp_ref, y2_ref, dinv_ref, w2_ref, b2_ref, out_ref):
    agg = aggp_ref[0, :N, :] + aggp_ref[1, :N, :] + y2_ref[:N, :]
    z = agg * dinv_ref[:N][:, None]
    logits = jnp.dot(z, w2_ref[...], preferred_element_type=jnp.float32)
    logits = logits + b2_ref[...][None, :]
    m = jnp.max(logits, axis=1, keepdims=True)
    s = logits - m
    lse = jnp.log(jnp.sum(jnp.exp(s), axis=1, keepdims=True))
    out_ref[...] = s - lse


_tc3 = pl.pallas_call(
    _tc3_body,
    out_shape=jax.ShapeDtypeStruct((N, NCLS), jnp.float32),
)


def kernel(x, edge_index, W1, b1, W2, b2):
    ei = edge_index.astype(jnp.int32)
    pad = jnp.full((EPAD - E,), N, jnp.int32)
    srcp = jnp.concatenate([ei[0], pad]).reshape(EPAD // CHUNK, CHUNK)
    dstp = jnp.concatenate([ei[1], pad]).reshape(EPAD // CHUNK, CHUNK)

    degp = _deg_kernel(dstp)
    y1, dinv = _tc1(x, W1, degp)
    agg1 = _agg_kernel(y1, srcp, dstp)
    y2 = _tc2(agg1, y1, dinv, b1)
    agg2 = _agg_kernel(y2, srcp, dstp)
    return _tc3(agg2, y2, dinv, W2, b2)


# speedup vs baseline: 1.0005x; 1.0005x over previous
import functools

import jax
import jax.numpy as jnp
from jax import lax
from jax.experimental import pallas as pl
from jax.experimental.pallas import tpu as pltpu
from jax.experimental.pallas import tpu_sc as plsc

N = 10000
NPAD = 10240
D = 128
H = 16
NCLS = 40
E = 320000
EPAD = 327680
CHUNK = 128
NWORKERS = 32
NCHUNK = EPAD // (NWORKERS * CHUNK)
NCH0 = 48
NCH1 = 112
NCHMAX = max(NCH0, NCH1)
RPT = NPAD // 16

_mesh = plsc.VectorSubcoreMesh(core_axis_name="c", subcore_axis_name="s")


@functools.partial(
    pl.kernel,
    mesh=_mesh,
    out_type=jax.ShapeDtypeStruct((2, NPAD), jnp.float32),
    scratch_types=[
        pltpu.VMEM((NCHMAX, CHUNK), jnp.int32),
        pltpu.VMEM((CHUNK,), jnp.float32),
        pltpu.VMEM((RPT,), jnp.float32),
        pltpu.VMEM_SHARED((NPAD,), jnp.float32),
        pltpu.SemaphoreType.DMA,
    ],
)
def _deg_kernel(dst_hbm, out_hbm, idx_v, ones_v, zbuf_v, deg_sh, dsem):
    ci = lax.axis_index("c")
    si = lax.axis_index("s")
    nch = jnp.where(ci == 0, NCH0, NCH1)
    base = jnp.where(ci == 0, si * NCH0, 16 * NCH0 + si * NCH1)

    def fill_ones(i, _):
        ones_v[pl.ds(i * 16, 16)] = jnp.ones((16,), jnp.float32)
        return 0

    lax.fori_loop(0, CHUNK // 16, fill_ones, 0)

    def fill_zeros(i, _):
        zbuf_v[pl.ds(i * 16, 16)] = jnp.zeros((16,), jnp.float32)
        return 0

    lax.fori_loop(0, RPT // 16, fill_zeros, 0)
    pltpu.sync_copy(zbuf_v, deg_sh.at[pl.ds(si * RPT, RPT)])
    plsc.subcore_barrier()

    pltpu.sync_copy(dst_hbm.at[pl.ds(base, NCHMAX)], idx_v)

    def group(g, _):
        for b in range(16):
            pltpu.async_copy(ones_v, deg_sh.at[idx_v.at[g * 16 + b]], dsem,
                             add=True)
        for b in range(16):
            pltpu.make_async_copy(ones_v, deg_sh.at[idx_v.at[g * 16 + b]],
                                  dsem).wait()
        return 0

    lax.fori_loop(0, nch // 16, group, 0)
    plsc.subcore_barrier()
    pltpu.sync_copy(deg_sh.at[pl.ds(si * RPT, RPT)],
                    out_hbm.at[ci, pl.ds(si * RPT, RPT)])


@functools.partial(
    pl.kernel,
    mesh=_mesh,
    compiler_params=pltpu.CompilerParams(use_tc_tiling_on_sc=False),
    out_type=jax.ShapeDtypeStruct((2, NPAD, H), jnp.float32),
    scratch_types=[
        pltpu.VMEM((NCHMAX, CHUNK), jnp.int32),
        pltpu.VMEM((NCHMAX, CHUNK), jnp.int32),
        pltpu.VMEM((16, CHUNK, H), jnp.float32),
        pltpu.VMEM((CHUNK, H), jnp.float32),
        pltpu.VMEM_SHARED((NPAD, H), jnp.float32),
        [pltpu.SemaphoreType.DMA] * 8,
        [pltpu.SemaphoreType.DMA] * 8,
    ],
)
def _agg_kernel(y_hbm, src_hbm, dst_hbm, out_hbm,
                sidx, didx, rows, zbuf, acc_sh, gsem, ssem):
    ci = lax.axis_index("c")
    si = lax.axis_index("s")
    nch = jnp.where(ci == 0, NCH0, NCH1)
    base = jnp.where(ci == 0, si * NCH0, 16 * NCH0 + si * NCH1)

    idx_cp0 = pltpu.async_copy(src_hbm.at[pl.ds(base, NCHMAX)], sidx,
                               gsem[0])
    idx_cp1 = pltpu.async_copy(dst_hbm.at[pl.ds(base, NCHMAX)], didx,
                               gsem[1])

    with jax.named_scope("agg_zero"):
        def fill_zeros(i, _):
            zbuf[i, :] = jnp.zeros((16,), jnp.float32)
            return 0

        lax.fori_loop(0, CHUNK, fill_zeros, 0)

        def zero_slice(i, _):
            pltpu.sync_copy(zbuf,
                            acc_sh.at[pl.ds(si * RPT + i * CHUNK, CHUNK)])
            return 0

        lax.fori_loop(0, RPT // CHUNK, zero_slice, 0)
        idx_cp0.wait()
        idx_cp1.wait()
        plsc.subcore_barrier()

    sc_main = jax.named_scope("agg_main")
    sc_main.__enter__()
    for b in range(8):
        pltpu.async_copy(y_hbm.at[sidx.at[b]], rows.at[b], gsem[b])

    def group(g, _):
        for b in range(16):
            k = g * 16 + b
            sb = b % 8

            @pl.when(k >= 8)
            def _wait_scatter():
                pltpu.make_async_copy(rows.at[(b + 8) % 16],
                                      acc_sh.at[didx.at[k - 8]],
                                      ssem[sb]).wait()

            pltpu.make_async_copy(y_hbm.at[sidx.at[k]], rows.at[b],
                                  gsem[sb]).wait()

            @pl.when(k + 8 < nch)
            def _issue_gather():
                pltpu.async_copy(y_hbm.at[sidx.at[k + 8]],
                                 rows.at[(b + 8) % 16], gsem[sb])

            pltpu.async_copy(rows.at[b], acc_sh.at[didx.at[k]], ssem[sb],
                             add=True)
        return 0

    lax.fori_loop(0, nch // 16, group, 0)
    for b in range(8):
        k = nch - 8 + b
        pltpu.make_async_copy(rows.at[(8 + b) % 16], acc_sh.at[didx.at[k]],
                              ssem[b]).wait()
    plsc.subcore_barrier()
    sc_main.__exit__(None, None, None)
    with jax.named_scope("agg_wb"):
        pltpu.sync_copy(acc_sh.at[pl.ds(si * RPT, RPT)],
                        out_hbm.at[ci, pl.ds(si * RPT, RPT)])


def _tc1_body(x_ref, w1_ref, degp_ref, y1_ref, dinv_ref):
    deg = degp_ref[0, :] + degp_ref[1, :] + 1.0
    dinv = lax.rsqrt(deg)
    dinv_ref[...] = dinv
    xw = jnp.dot(x_ref[...], w1_ref[...], preferred_element_type=jnp.float32)
    y1_ref[:N, :] = xw * dinv[:N, None]
    y1_ref[N:, :] = jnp.zeros((NPAD - N, H), jnp.float32)


_tc1 = pl.pallas_call(
    _tc1_body,
    out_shape=(jax.ShapeDtypeStruct((NPAD, H), jnp.float32),
               jax.ShapeDtypeStruct((NPAD,), jnp.float32)),
)


def _tc2_body(aggp_ref, y1_ref, dinv_ref, b1_ref, y2_ref):
    agg = aggp_ref[0] + aggp_ref[1] + y1_ref[...]
    dinv = dinv_ref[...][:, None]
    h = jnp.maximum(agg * dinv + b1_ref[...][None, :], 0.0)
    y2_ref[...] = h * dinv


_tc2 = pl.pallas_call(
    _tc2_body,
    out_shape=jax.ShapeDtypeStruct((NPAD, H), jnp.float32),
)


def _tc3_body(aggp_ref, y2_ref, dinv_ref, w2_ref, b2_ref, out_ref):
    agg = aggp_ref[0, :N, :] + aggp_ref[1, :N, :] + y2_ref[:N, :]
    z = agg * dinv_ref[:N][:, None]
    logits = jnp.dot(z, w2_ref[...], preferred_element_type=jnp.float32)
    logits = logits + b2_ref[...][None, :]
    m = jnp.max(logits, axis=1, keepdims=True)
    s = logits - m
    lse = jnp.log(jnp.sum(jnp.exp(s), axis=1, keepdims=True))
    out_ref[...] = s - lse


_tc3 = pl.pallas_call(
    _tc3_body,
    out_shape=jax.ShapeDtypeStruct((N, NCLS), jnp.float32),
)


def kernel(x, edge_index, W1, b1, W2, b2):
    ei = edge_index.astype(jnp.int32)
    pad = jnp.full((EPAD - E,), N, jnp.int32)
    srcp = jnp.concatenate([ei[0], pad]).reshape(EPAD // CHUNK, CHUNK)
    dstp = jnp.concatenate([ei[1], pad]).reshape(EPAD // CHUNK, CHUNK)

    degp = _deg_kernel(dstp)
    y1, dinv = _tc1(x, W1, degp)
    agg1 = _agg_kernel(y1, srcp, dstp)
    y2 = _tc2(agg1, y1, dinv, b1)
    agg2 = _agg_kernel(y2, srcp, dstp)
    return _tc3(agg2, y2, dinv, W2, b2)


# speedup vs baseline: 1.0522x; 1.0516x over previous
import functools

import jax
import jax.numpy as jnp
from jax import lax
from jax.experimental import pallas as pl
from jax.experimental.pallas import tpu as pltpu
from jax.experimental.pallas import tpu_sc as plsc

N = 10000
NPAD = 10240
D = 128
H = 16
NCLS = 40
E = 320000
EPAD = 327680
CHUNK = 128
NWORKERS = 32
NCHUNK = EPAD // (NWORKERS * CHUNK)
NCH0 = 112
NCH1 = 48
NCHMAX = max(NCH0, NCH1)
RPT = NPAD // 16

_mesh = plsc.VectorSubcoreMesh(core_axis_name="c", subcore_axis_name="s")


@functools.partial(
    pl.kernel,
    mesh=_mesh,
    out_type=jax.ShapeDtypeStruct((2, NPAD), jnp.float32),
    scratch_types=[
        pltpu.VMEM((NCHMAX, CHUNK), jnp.int32),
        pltpu.VMEM((CHUNK,), jnp.float32),
        pltpu.VMEM((RPT,), jnp.float32),
        pltpu.VMEM_SHARED((NPAD,), jnp.float32),
        pltpu.SemaphoreType.DMA,
    ],
)
def _deg_kernel(dst_hbm, out_hbm, idx_v, ones_v, zbuf_v, deg_sh, dsem):
    ci = lax.axis_index("c")
    si = lax.axis_index("s")
    nch = jnp.where(ci == 0, NCH0, NCH1)
    base = jnp.where(ci == 0, si * NCH0, 16 * NCH0 + si * NCH1)

    def fill_ones(i, _):
        ones_v[pl.ds(i * 16, 16)] = jnp.ones((16,), jnp.float32)
        return 0

    lax.fori_loop(0, CHUNK // 16, fill_ones, 0)

    def fill_zeros(i, _):
        zbuf_v[pl.ds(i * 16, 16)] = jnp.zeros((16,), jnp.float32)
        return 0

    lax.fori_loop(0, RPT // 16, fill_zeros, 0)
    pltpu.sync_copy(zbuf_v, deg_sh.at[pl.ds(si * RPT, RPT)])
    plsc.subcore_barrier()

    pltpu.sync_copy(dst_hbm.at[pl.ds(base, NCHMAX)], idx_v)

    def group(g, _):
        for b in range(16):
            pltpu.async_copy(ones_v, deg_sh.at[idx_v.at[g * 16 + b]], dsem,
                             add=True)
        for b in range(16):
            pltpu.make_async_copy(ones_v, deg_sh.at[idx_v.at[g * 16 + b]],
                                  dsem).wait()
        return 0

    lax.fori_loop(0, nch // 16, group, 0)
    plsc.subcore_barrier()
    pltpu.sync_copy(deg_sh.at[pl.ds(si * RPT, RPT)],
                    out_hbm.at[ci, pl.ds(si * RPT, RPT)])


@functools.partial(
    pl.kernel,
    mesh=_mesh,
    compiler_params=pltpu.CompilerParams(use_tc_tiling_on_sc=False),
    out_type=jax.ShapeDtypeStruct((2, NPAD, H), jnp.float32),
    scratch_types=[
        pltpu.VMEM((NCHMAX, CHUNK), jnp.int32),
        pltpu.VMEM((NCHMAX, CHUNK), jnp.int32),
        pltpu.VMEM((16, CHUNK, H), jnp.float32),
        pltpu.VMEM((CHUNK, H), jnp.float32),
        pltpu.VMEM_SHARED((NPAD, H), jnp.float32),
        [pltpu.SemaphoreType.DMA] * 8,
        [pltpu.SemaphoreType.DMA] * 8,
    ],
)
def _agg_kernel(y_hbm, src_hbm, dst_hbm, out_hbm,
                sidx, didx, rows, zbuf, acc_sh, gsem, ssem):
    ci = lax.axis_index("c")
    si = lax.axis_index("s")
    nch = jnp.where(ci == 0, NCH0, NCH1)
    base = jnp.where(ci == 0, si * NCH0, 16 * NCH0 + si * NCH1)

    idx_cp0 = pltpu.async_copy(src_hbm.at[pl.ds(base, NCHMAX)], sidx,
                               gsem[0])
    idx_cp1 = pltpu.async_copy(dst_hbm.at[pl.ds(base, NCHMAX)], didx,
                               gsem[1])

    def fill_zeros(i, _):
        zbuf[i, :] = jnp.zeros((16,), jnp.float32)
        return 0

    lax.fori_loop(0, CHUNK, fill_zeros, 0)

    def zero_slice(i, _):
        pltpu.sync_copy(zbuf, acc_sh.at[pl.ds(si * RPT + i * CHUNK, CHUNK)])
        return 0

    lax.fori_loop(0, RPT // CHUNK, zero_slice, 0)
    idx_cp0.wait()
    idx_cp1.wait()
    plsc.subcore_barrier()

    for b in range(8):
        pltpu.async_copy(y_hbm.at[sidx.at[b]], rows.at[b], gsem[b])

    def group(g, _):
        for b in range(16):
            k = g * 16 + b
            sb = b % 8

            @pl.when(k >= 8)
            def _wait_scatter():
                pltpu.make_async_copy(rows.at[(b + 8) % 16],
                                      acc_sh.at[didx.at[k - 8]],
                                      ssem[sb]).wait()

            pltpu.make_async_copy(y_hbm.at[sidx.at[k]], rows.at[b],
                                  gsem[sb]).wait()

            @pl.when(k + 8 < nch)
            def _issue_gather():
                pltpu.async_copy(y_hbm.at[sidx.at[k + 8]],
                                 rows.at[(b + 8) % 16], gsem[sb])

            pltpu.async_copy(rows.at[b], acc_sh.at[didx.at[k]], ssem[sb],
                             add=True)
        return 0

    lax.fori_loop(0, nch // 16, group, 0)
    for b in range(8):
        k = nch - 8 + b
        pltpu.make_async_copy(rows.at[(8 + b) % 16], acc_sh.at[didx.at[k]],
                              ssem[b]).wait()
    plsc.subcore_barrier()
    pltpu.sync_copy(acc_sh.at[pl.ds(si * RPT, RPT)],
                    out_hbm.at[ci, pl.ds(si * RPT, RPT)])


def _tc1_body(x_ref, w1_ref, degp_ref, y1_ref, dinv_ref):
    deg = degp_ref[0, :] + degp_ref[1, :] + 1.0
    dinv = lax.rsqrt(deg)
    dinv_ref[...] = dinv
    xw = jnp.dot(x_ref[...], w1_ref[...], preferred_element_type=jnp.float32)
    y1_ref[:N, :] = xw * dinv[:N, None]
    y1_ref[N:, :] = jnp.zeros((NPAD - N, H), jnp.float32)


_tc1 = pl.pallas_call(
    _tc1_body,
    out_shape=(jax.ShapeDtypeStruct((NPAD, H), jnp.float32),
               jax.ShapeDtypeStruct((NPAD,), jnp.float32)),
)


def _tc2_body(aggp_ref, y1_ref, dinv_ref, b1_ref, y2_ref):
    agg = aggp_ref[0] + aggp_ref[1] + y1_ref[...]
    dinv = dinv_ref[...][:, None]
    h = jnp.maximum(agg * dinv + b1_ref[...][None, :], 0.0)
    y2_ref[...] = h * dinv


_tc2 = pl.pallas_call(
    _tc2_body,
    out_shape=jax.ShapeDtypeStruct((NPAD, H), jnp.float32),
)


def _tc3_body(aggp_ref, y2_ref, dinv_ref, w2_ref, b2_ref, out_ref):
    agg = aggp_ref[0, :N, :] + aggp_ref[1, :N, :] + y2_ref[:N, :]
    z = agg * dinv_ref[:N][:, None]
    logits = jnp.dot(z, w2_ref[...], preferred_element_type=jnp.float32)
    logits = logits + b2_ref[...][None, :]
    m = jnp.max(logits, axis=1, keepdims=True)
    s = logits - m
    lse = jnp.log(jnp.sum(jnp.exp(s), axis=1, keepdims=True))
    out_ref[...] = s - lse


_tc3 = pl.pallas_call(
    _tc3_body,
    out_shape=jax.ShapeDtypeStruct((N, NCLS), jnp.float32),
)


def kernel(x, edge_index, W1, b1, W2, b2):
    ei = edge_index.astype(jnp.int32)
    pad = jnp.full((EPAD - E,), N, jnp.int32)
    srcp = jnp.concatenate([ei[0], pad]).reshape(EPAD // CHUNK, CHUNK)
    dstp = jnp.concatenate([ei[1], pad]).reshape(EPAD // CHUNK, CHUNK)

    degp = _deg_kernel(dstp)
    y1, dinv = _tc1(x, W1, degp)
    agg1 = _agg_kernel(y1, srcp, dstp)
    y2 = _tc2(agg1, y1, dinv, b1)
    agg2 = _agg_kernel(y2, srcp, dstp)
    return _tc3(agg2, y2, dinv, W2, b2)
